# Initial kernel scaffold; baseline (speedup 1.0000x reference)
import jax
import jax.numpy as jnp
from jax.experimental import pallas as pl


def kernel(x, edge_index, W1, W2, W3, lin1_w, lin1_b, ln_g, ln_b, lin2_w, lin2_b):
    raise NotImplementedError("write your pallas kernel here")


# speedup vs baseline: 5.4127x; 5.4127x over previous
import functools

import jax
import jax.numpy as jnp
from jax import lax
from jax.experimental import pallas as pl
from jax.experimental.pallas import tpu as pltpu
from jax.experimental.pallas import tpu_sc as plsc

N_NODES = 10000
N_EDGES = 320000
D = 128
HIDDEN = 256
OUT_D = 64
NC = 2
NS = 16
NW = NC * NS
E_PER_W = N_EDGES // NW
CHUNK = 125
NCHUNK = E_PER_W // CHUNK
GRP = 8
NGRP = NCHUNK // GRP
N_PAD = 10240
ROWS_PER_TILE = N_PAD // NS


def _sc_mesh():
    return plsc.VectorSubcoreMesh(
        core_axis_name="c", subcore_axis_name="s", num_cores=NC, num_subcores=NS
    )


def _make_sc_agg():
    scratch = [
        pltpu.VMEM_SHARED((N_PAD, D), jnp.float32),
        pltpu.VMEM((GRP, CHUNK), jnp.int32),
        pltpu.VMEM((GRP, CHUNK), jnp.int32),
        pltpu.VMEM((CHUNK, D), jnp.float32),
        pltpu.SemaphoreType.DMA,
    ]

    def body(table, src3, dst3, zrows, psum, acc_sh, src_v, dst_v, rows_v, sem):
        cid = lax.axis_index("c")
        sid = lax.axis_index("s")
        wid = cid * NS + sid
        r0 = sid * ROWS_PER_TILE

        pltpu.sync_copy(zrows.at[pl.ds(r0, ROWS_PER_TILE)],
                        acc_sh.at[pl.ds(r0, ROWS_PER_TILE)])
        plsc.subcore_barrier()

        def group_body(g, carry):
            pltpu.sync_copy(src3.at[wid, pl.ds(g * GRP, GRP)], src_v)
            pltpu.sync_copy(dst3.at[wid, pl.ds(g * GRP, GRP)], dst_v)

            def chunk_body(j, c2):
                pltpu.async_copy(table.at[src_v.at[j]], rows_v, sem).wait()
                pltpu.sync_copy(rows_v, acc_sh.at[dst_v.at[j]], add=True)
                return c2

            lax.fori_loop(0, GRP, chunk_body, 0)
            return carry

        lax.fori_loop(0, NGRP, group_body, 0)
        plsc.subcore_barrier()

        pltpu.sync_copy(acc_sh.at[pl.ds(r0, ROWS_PER_TILE)],
                        psum.at[cid, pl.ds(r0, ROWS_PER_TILE)])

    return pl.kernel(
        body, out_type=jax.ShapeDtypeStruct((NC, N_PAD, D), jnp.float32),
        mesh=_sc_mesh(), scratch_types=scratch,
    )


_sc_agg = _make_sc_agg()

BLK = 1000


def _tc_layer_body(p0_ref, p1_ref, inv_ref, w_ref, o_ref):
    m = (p0_ref[...] + p1_ref[...]) * inv_ref[...]
    o_ref[...] = lax.dot_general(
        m, w_ref[...], (((1,), (1,)), ((), ())),
        preferred_element_type=jnp.float32)


def _tc_layer(p0, p1, inv, W):
    grid = (N_NODES // BLK,)
    return pl.pallas_call(
        _tc_layer_body,
        grid=grid,
        in_specs=[
            pl.BlockSpec((BLK, D), lambda i: (i, 0)),
            pl.BlockSpec((BLK, D), lambda i: (i, 0)),
            pl.BlockSpec((BLK, 1), lambda i: (i, 0)),
            pl.BlockSpec((D, D), lambda i: (0, 0)),
        ],
        out_specs=pl.BlockSpec((BLK, D), lambda i: (i, 0)),
        out_shape=jax.ShapeDtypeStruct((N_NODES, D), jnp.float32),
    )(p0, p1, inv, W)


def _tc_final_body(x_ref, z1_ref, z2_ref, p30_ref, p31_ref, inv_ref, w3_ref,
                   l0_ref, l1_ref, l2_ref, l3_ref, b1_ref, g_ref, bb_ref,
                   l2w_ref, b2_ref, o_ref):
    dn = (((1,), (1,)), ((), ()))
    z3 = lax.dot_general(
        (p30_ref[...] + p31_ref[...]) * inv_ref[...], w3_ref[...], dn,
        preferred_element_type=jnp.float32)
    h = (lax.dot_general(x_ref[...], l0_ref[...], dn, preferred_element_type=jnp.float32)
         + lax.dot_general(z1_ref[...], l1_ref[...], dn, preferred_element_type=jnp.float32)
         + lax.dot_general(z2_ref[...], l2_ref[...], dn, preferred_element_type=jnp.float32)
         + lax.dot_general(z3, l3_ref[...], dn, preferred_element_type=jnp.float32)
         + b1_ref[...])
    mu = jnp.mean(h, axis=1, keepdims=True)
    var = jnp.mean((h - mu) ** 2, axis=1, keepdims=True)
    hn = (h - mu) * lax.rsqrt(var + 1e-5) * g_ref[...] + bb_ref[...]
    hr = jnp.maximum(hn, 0.0)
    o_ref[...] = lax.dot_general(
        hr, l2w_ref[...], dn, preferred_element_type=jnp.float32) + b2_ref[...]


def _tc_final(x, z1, z2, p30, p31, inv, W3, l1w, lin1_b, ln_g, ln_b,
              lin2_w, lin2_b):
    grid = (N_NODES // BLK,)
    l0, l1, l2, l3 = (l1w[:, 0:D], l1w[:, D:2 * D],
                      l1w[:, 2 * D:3 * D], l1w[:, 3 * D:4 * D])
    row = pl.BlockSpec((BLK, D), lambda i: (i, 0))
    full = lambda a, b: pl.BlockSpec((a, b), lambda i: (0, 0))
    return pl.pallas_call(
        _tc_final_body,
        grid=grid,
        in_specs=[
            row, row, row, row, row,
            pl.BlockSpec((BLK, 1), lambda i: (i, 0)),
            full(D, D),
            full(HIDDEN, D), full(HIDDEN, D), full(HIDDEN, D), full(HIDDEN, D),
            full(1, HIDDEN), full(1, HIDDEN), full(1, HIDDEN),
            full(OUT_D, HIDDEN), full(1, OUT_D),
        ],
        out_specs=pl.BlockSpec((BLK, OUT_D), lambda i: (i, 0)),
        out_shape=jax.ShapeDtypeStruct((N_NODES, OUT_D), jnp.float32),
    )(x, z1, z2, p30, p31, inv, W3, l0, l1, l2, l3,
      lin1_b.reshape(1, HIDDEN), ln_g.reshape(1, HIDDEN),
      ln_b.reshape(1, HIDDEN), lin2_w, lin2_b.reshape(1, OUT_D))


def kernel(x, edge_index, W1, W2, W3, lin1_w, lin1_b, ln_g, ln_b,
           lin2_w, lin2_b):
    ei = edge_index.astype(jnp.int32)
    src3 = ei[0].reshape(NW, NCHUNK, CHUNK)
    dst3 = ei[1].reshape(NW, NCHUNK, CHUNK)
    zrows = jnp.zeros((N_PAD, D), jnp.float32)
    ones_tab = jnp.ones((N_NODES, D), jnp.float32)

    psc = _sc_agg(ones_tab, src3, dst3, zrows)
    cnt = psc[0, :N_NODES, 0:1] + psc[1, :N_NODES, 0:1]
    inv = 1.0 / jnp.maximum(cnt, 1.0)

    psum1 = _sc_agg(x, src3, dst3, zrows)
    z1 = _tc_layer(psum1[0, :N_NODES], psum1[1, :N_NODES], inv, W1)
    psum2 = _sc_agg(z1, src3, dst3, zrows)
    z2 = _tc_layer(psum2[0, :N_NODES], psum2[1, :N_NODES], inv, W2)
    psum3 = _sc_agg(z2, src3, dst3, zrows)
    return _tc_final(x, z1, z2, psum3[0, :N_NODES], psum3[1, :N_NODES], inv,
                     W3, lin1_w, lin1_b, ln_g, ln_b, lin2_w, lin2_b)


# speedup vs baseline: 8.4449x; 1.5602x over previous
import functools

import jax
import jax.numpy as jnp
from jax import lax
from jax.experimental import pallas as pl
from jax.experimental.pallas import tpu as pltpu
from jax.experimental.pallas import tpu_sc as plsc

N_NODES = 10000
N_EDGES = 320000
D = 128
HIDDEN = 256
OUT_D = 64
NC = 2
NS = 16
NW = NC * NS
E_PER_W = N_EDGES // NW
CHUNK = 100
GRP = 20
NGRP = 5
N_PAD = 10240
ROWS_PER_TILE = N_PAD // NS


def _sc_mesh():
    return plsc.VectorSubcoreMesh(
        core_axis_name="c", subcore_axis_name="s", num_cores=NC, num_subcores=NS
    )


def _make_sc_agg():
    scratch = [
        pltpu.VMEM_SHARED((N_PAD, D), jnp.float32),
        pltpu.VMEM((GRP, CHUNK), jnp.int32),
        pltpu.VMEM((GRP, CHUNK), jnp.int32),
        pltpu.VMEM((CHUNK, D), jnp.float32),
        pltpu.VMEM((CHUNK, D), jnp.float32),
        pltpu.SemaphoreType.DMA,
        pltpu.SemaphoreType.DMA,
        pltpu.SemaphoreType.DMA,
        pltpu.SemaphoreType.DMA,
    ]

    def body(table, src4, dst4, zrows, psum, acc_sh, src_v, dst_v,
             rows0, rows1, g0, g1, s0, s1):
        cid = lax.axis_index("c")
        sid = lax.axis_index("s")
        wid = cid * NS + sid
        r0 = sid * ROWS_PER_TILE
        rows = (rows0, rows1)
        gsem = (g0, g1)
        ssem = (s0, s1)

        pltpu.sync_copy(zrows.at[pl.ds(r0, ROWS_PER_TILE)],
                        acc_sh.at[pl.ds(r0, ROWS_PER_TILE)])
        plsc.subcore_barrier()

        def group_body(g, carry):
            pltpu.sync_copy(src4.at[wid, g], src_v)
            pltpu.sync_copy(dst4.at[wid, g], dst_v)
            gathers = [None, None]
            scatters = [None, None]
            for c in range(GRP):
                b = c % 2
                if scatters[b] is not None:
                    scatters[b].wait()
                gathers[b] = pltpu.async_copy(
                    table.at[src_v.at[c]], rows[b], gsem[b])
                if c >= 1:
                    gathers[1 - b].wait()
                    scatters[1 - b] = pltpu.async_copy(
                        rows[1 - b], acc_sh.at[dst_v.at[c - 1]], ssem[1 - b],
                        add=True)
            gathers[1].wait()
            scatters[1] = pltpu.async_copy(
                rows[1], acc_sh.at[dst_v.at[GRP - 1]], ssem[1], add=True)
            scatters[0].wait()
            scatters[1].wait()
            return carry

        lax.fori_loop(0, NGRP, group_body, 0)
        plsc.subcore_barrier()

        pltpu.sync_copy(acc_sh.at[pl.ds(r0, ROWS_PER_TILE)],
                        psum.at[cid, pl.ds(r0, ROWS_PER_TILE)])

    return pl.kernel(
        body, out_type=jax.ShapeDtypeStruct((NC, N_PAD, D), jnp.float32),
        mesh=_sc_mesh(), scratch_types=scratch,
    )


def _make_sc_count():
    scratch = [
        pltpu.VMEM_SHARED((N_PAD, D), jnp.float32),
        pltpu.VMEM((GRP, CHUNK), jnp.int32),
        pltpu.VMEM((CHUNK, D), jnp.float32),
        pltpu.SemaphoreType.DMA,
    ]

    def body(ones_tab, dst4, zrows, psum, acc_sh, dst_v, ones_v, sem):
        cid = lax.axis_index("c")
        sid = lax.axis_index("s")
        wid = cid * NS + sid
        r0 = sid * ROWS_PER_TILE

        pltpu.sync_copy(zrows.at[pl.ds(r0, ROWS_PER_TILE)],
                        acc_sh.at[pl.ds(r0, ROWS_PER_TILE)])
        pltpu.sync_copy(ones_tab.at[pl.ds(0, CHUNK)], ones_v)
        plsc.subcore_barrier()

        def group_body(g, carry):
            pltpu.sync_copy(dst4.at[wid, g], dst_v)
            descs = [
                pltpu.async_copy(ones_v, acc_sh.at[dst_v.at[c]], sem, add=True)
                for c in range(GRP)
            ]
            for d in descs:
                d.wait()
            return carry

        lax.fori_loop(0, NGRP, group_body, 0)
        plsc.subcore_barrier()
        pltpu.sync_copy(acc_sh.at[pl.ds(r0, ROWS_PER_TILE)],
                        psum.at[cid, pl.ds(r0, ROWS_PER_TILE)])

    return pl.kernel(
        body, out_type=jax.ShapeDtypeStruct((NC, N_PAD, D), jnp.float32),
        mesh=_sc_mesh(), scratch_types=scratch,
    )


_sc_agg = _make_sc_agg()
_sc_count = _make_sc_count()

BLK = 1000


def _tc_layer_body(p0_ref, p1_ref, inv_ref, w_ref, o_ref):
    m = (p0_ref[...] + p1_ref[...]) * inv_ref[...]
    o_ref[...] = lax.dot_general(
        m, w_ref[...], (((1,), (1,)), ((), ())),
        preferred_element_type=jnp.float32)


def _tc_layer(p0, p1, inv, W):
    grid = (N_NODES // BLK,)
    return pl.pallas_call(
        _tc_layer_body,
        grid=grid,
        in_specs=[
            pl.BlockSpec((BLK, D), lambda i: (i, 0)),
            pl.BlockSpec((BLK, D), lambda i: (i, 0)),
            pl.BlockSpec((BLK, 1), lambda i: (i, 0)),
            pl.BlockSpec((D, D), lambda i: (0, 0)),
        ],
        out_specs=pl.BlockSpec((BLK, D), lambda i: (i, 0)),
        out_shape=jax.ShapeDtypeStruct((N_NODES, D), jnp.float32),
    )(p0, p1, inv, W)


def _tc_final_body(x_ref, z1_ref, z2_ref, p30_ref, p31_ref, inv_ref, w3_ref,
                   l0_ref, l1_ref, l2_ref, l3_ref, b1_ref, g_ref, bb_ref,
                   l2w_ref, b2_ref, o_ref):
    dn = (((1,), (1,)), ((), ()))
    z3 = lax.dot_general(
        (p30_ref[...] + p31_ref[...]) * inv_ref[...], w3_ref[...], dn,
        preferred_element_type=jnp.float32)
    h = (lax.dot_general(x_ref[...], l0_ref[...], dn, preferred_element_type=jnp.float32)
         + lax.dot_general(z1_ref[...], l1_ref[...], dn, preferred_element_type=jnp.float32)
         + lax.dot_general(z2_ref[...], l2_ref[...], dn, preferred_element_type=jnp.float32)
         + lax.dot_general(z3, l3_ref[...], dn, preferred_element_type=jnp.float32)
         + b1_ref[...])
    mu = jnp.mean(h, axis=1, keepdims=True)
    var = jnp.mean((h - mu) ** 2, axis=1, keepdims=True)
    hn = (h - mu) * lax.rsqrt(var + 1e-5) * g_ref[...] + bb_ref[...]
    hr = jnp.maximum(hn, 0.0)
    o_ref[...] = lax.dot_general(
        hr, l2w_ref[...], dn, preferred_element_type=jnp.float32) + b2_ref[...]


def _tc_final(x, z1, z2, p30, p31, inv, W3, l1w, lin1_b, ln_g, ln_b,
              lin2_w, lin2_b):
    grid = (N_NODES // BLK,)
    l0, l1, l2, l3 = (l1w[:, 0:D], l1w[:, D:2 * D],
                      l1w[:, 2 * D:3 * D], l1w[:, 3 * D:4 * D])
    row = pl.BlockSpec((BLK, D), lambda i: (i, 0))
    full = lambda a, b: pl.BlockSpec((a, b), lambda i: (0, 0))
    return pl.pallas_call(
        _tc_final_body,
        grid=grid,
        in_specs=[
            row, row, row, row, row,
            pl.BlockSpec((BLK, 1), lambda i: (i, 0)),
            full(D, D),
            full(HIDDEN, D), full(HIDDEN, D), full(HIDDEN, D), full(HIDDEN, D),
            full(1, HIDDEN), full(1, HIDDEN), full(1, HIDDEN),
            full(OUT_D, HIDDEN), full(1, OUT_D),
        ],
        out_specs=pl.BlockSpec((BLK, OUT_D), lambda i: (i, 0)),
        out_shape=jax.ShapeDtypeStruct((N_NODES, OUT_D), jnp.float32),
    )(x, z1, z2, p30, p31, inv, W3, l0, l1, l2, l3,
      lin1_b.reshape(1, HIDDEN), ln_g.reshape(1, HIDDEN),
      ln_b.reshape(1, HIDDEN), lin2_w, lin2_b.reshape(1, OUT_D))


def kernel(x, edge_index, W1, W2, W3, lin1_w, lin1_b, ln_g, ln_b,
           lin2_w, lin2_b):
    ei = edge_index.astype(jnp.int32)
    src4 = ei[0].reshape(NW, NGRP, GRP, CHUNK)
    dst4 = ei[1].reshape(NW, NGRP, GRP, CHUNK)
    zrows = jnp.zeros((N_PAD, D), jnp.float32)
    ones_tab = jnp.ones((CHUNK, D), jnp.float32)

    psc = _sc_count(ones_tab, dst4, zrows)
    cnt = psc[0, :N_NODES, 0:1] + psc[1, :N_NODES, 0:1]
    inv = 1.0 / jnp.maximum(cnt, 1.0)

    psum1 = _sc_agg(x, src4, dst4, zrows)
    z1 = _tc_layer(psum1[0, :N_NODES], psum1[1, :N_NODES], inv, W1)
    psum2 = _sc_agg(z1, src4, dst4, zrows)
    z2 = _tc_layer(psum2[0, :N_NODES], psum2[1, :N_NODES], inv, W2)
    psum3 = _sc_agg(z2, src4, dst4, zrows)
    return _tc_final(x, z1, z2, psum3[0, :N_NODES], psum3[1, :N_NODES], inv,
                     W3, lin1_w, lin1_b, ln_g, ln_b, lin2_w, lin2_b)


# speedup vs baseline: 8.5949x; 1.0178x over previous
import functools

import jax
import jax.numpy as jnp
from jax import lax
from jax.experimental import pallas as pl
from jax.experimental.pallas import tpu as pltpu
from jax.experimental.pallas import tpu_sc as plsc

N_NODES = 10000
N_EDGES = 320000
D = 128
HIDDEN = 256
OUT_D = 64
NC = 2
NS = 16
NW = NC * NS
E_PER_W = N_EDGES // NW
CHUNK = 100
GRP = 20
NGRP = 5
N_PAD = 10240
ROWS_PER_TILE = N_PAD // NS


def _sc_mesh():
    return plsc.VectorSubcoreMesh(
        core_axis_name="c", subcore_axis_name="s", num_cores=NC, num_subcores=NS
    )


def _make_sc_agg():
    scratch = [
        pltpu.VMEM_SHARED((N_PAD, D), jnp.float32),
        pltpu.VMEM((GRP, CHUNK), jnp.int32),
        pltpu.VMEM((GRP, CHUNK), jnp.int32),
        pltpu.VMEM((CHUNK, D), jnp.float32),
        pltpu.VMEM((CHUNK, D), jnp.float32),
        pltpu.SemaphoreType.DMA,
        pltpu.SemaphoreType.DMA,
        pltpu.SemaphoreType.DMA,
        pltpu.SemaphoreType.DMA,
    ]

    def body(table, src4, dst4, zrows, psum, acc_sh, src_v, dst_v,
             rows0, rows1, g0, g1, s0, s1):
        cid = lax.axis_index("c")
        sid = lax.axis_index("s")
        wid = cid * NS + sid
        r0 = sid * ROWS_PER_TILE
        rows = (rows0, rows1)
        gsem = (g0, g1)
        ssem = (s0, s1)

        pltpu.sync_copy(zrows.at[pl.ds(r0, ROWS_PER_TILE)],
                        acc_sh.at[pl.ds(r0, ROWS_PER_TILE)])
        plsc.subcore_barrier()

        def group_body(g, carry):
            pltpu.sync_copy(src4.at[wid, g], src_v)
            pltpu.sync_copy(dst4.at[wid, g], dst_v)
            gathers = [None, None]
            scatters = [None, None]
            for c in range(GRP):
                b = c % 2
                if scatters[b] is not None:
                    scatters[b].wait()
                gathers[b] = pltpu.async_copy(
                    table.at[src_v.at[c]], rows[b], gsem[b])
                if c >= 1:
                    gathers[1 - b].wait()
                    scatters[1 - b] = pltpu.async_copy(
                        rows[1 - b], acc_sh.at[dst_v.at[c - 1]], ssem[1 - b],
                        add=True)
            gathers[1].wait()
            scatters[1] = pltpu.async_copy(
                rows[1], acc_sh.at[dst_v.at[GRP - 1]], ssem[1], add=True)
            scatters[0].wait()
            scatters[1].wait()
            return carry

        lax.fori_loop(0, NGRP, group_body, 0)
        plsc.subcore_barrier()

        pltpu.sync_copy(acc_sh.at[pl.ds(r0, ROWS_PER_TILE)],
                        psum.at[cid, pl.ds(r0, ROWS_PER_TILE)])

    return pl.kernel(
        body, out_type=jax.ShapeDtypeStruct((NC, N_PAD, D), jnp.float32),
        mesh=_sc_mesh(), scratch_types=scratch,
    )


def _make_sc_count():
    scratch = [
        pltpu.VMEM_SHARED((N_PAD, D), jnp.float32),
        pltpu.VMEM((GRP, CHUNK), jnp.int32),
        pltpu.VMEM((CHUNK, D), jnp.float32),
        pltpu.SemaphoreType.DMA,
    ]

    def body(ones_tab, dst4, zrows, psum, acc_sh, dst_v, ones_v, sem):
        cid = lax.axis_index("c")
        sid = lax.axis_index("s")
        wid = cid * NS + sid
        r0 = sid * ROWS_PER_TILE

        pltpu.sync_copy(zrows.at[pl.ds(r0, ROWS_PER_TILE)],
                        acc_sh.at[pl.ds(r0, ROWS_PER_TILE)])
        pltpu.sync_copy(ones_tab.at[pl.ds(0, CHUNK)], ones_v)
        plsc.subcore_barrier()

        def group_body(g, carry):
            pltpu.sync_copy(dst4.at[wid, g], dst_v)
            descs = [
                pltpu.async_copy(ones_v, acc_sh.at[dst_v.at[c]], sem, add=True)
                for c in range(GRP)
            ]
            for d in descs:
                d.wait()
            return carry

        lax.fori_loop(0, NGRP, group_body, 0)
        plsc.subcore_barrier()
        pltpu.sync_copy(acc_sh.at[pl.ds(r0, ROWS_PER_TILE)],
                        psum.at[cid, pl.ds(r0, ROWS_PER_TILE)])

    return pl.kernel(
        body, out_type=jax.ShapeDtypeStruct((NC, N_PAD, D), jnp.float32),
        mesh=_sc_mesh(), scratch_types=scratch,
    )


_sc_agg = _make_sc_agg()
_sc_count = _make_sc_count()

BLK = 640


def _tc_layer_body(p0_ref, p1_ref, inv_ref, w_ref, o_ref):
    m = (p0_ref[0] + p1_ref[0]) * inv_ref[...]
    o_ref[...] = lax.dot_general(
        m, w_ref[...], (((1,), (1,)), ((), ())),
        preferred_element_type=jnp.float32)


def _tc_layer(psum, inv, W):
    grid = (N_PAD // BLK,)
    return pl.pallas_call(
        _tc_layer_body,
        grid=grid,
        in_specs=[
            pl.BlockSpec((1, BLK, D), lambda i: (0, i, 0)),
            pl.BlockSpec((1, BLK, D), lambda i: (1, i, 0)),
            pl.BlockSpec((BLK, 1), lambda i: (i, 0)),
            pl.BlockSpec((D, D), lambda i: (0, 0)),
        ],
        out_specs=pl.BlockSpec((BLK, D), lambda i: (i, 0)),
        out_shape=jax.ShapeDtypeStruct((N_PAD, D), jnp.float32),
    )(psum, psum, inv, W)


def _tc_final_body(x_ref, z1_ref, z2_ref, p30_ref, p31_ref, inv_ref, w3_ref,
                   l0_ref, l1_ref, l2_ref, l3_ref, b1_ref, g_ref, bb_ref,
                   l2w_ref, b2_ref, o_ref):
    dn = (((1,), (1,)), ((), ()))
    z3 = lax.dot_general(
        (p30_ref[0] + p31_ref[0]) * inv_ref[...], w3_ref[...], dn,
        preferred_element_type=jnp.float32)
    h = (lax.dot_general(x_ref[...], l0_ref[...], dn, preferred_element_type=jnp.float32)
         + lax.dot_general(z1_ref[...], l1_ref[...], dn, preferred_element_type=jnp.float32)
         + lax.dot_general(z2_ref[...], l2_ref[...], dn, preferred_element_type=jnp.float32)
         + lax.dot_general(z3, l3_ref[...], dn, preferred_element_type=jnp.float32)
         + b1_ref[...])
    mu = jnp.mean(h, axis=1, keepdims=True)
    var = jnp.mean((h - mu) ** 2, axis=1, keepdims=True)
    hn = (h - mu) * lax.rsqrt(var + 1e-5) * g_ref[...] + bb_ref[...]
    hr = jnp.maximum(hn, 0.0)
    o_ref[...] = lax.dot_general(
        hr, l2w_ref[...], dn, preferred_element_type=jnp.float32) + b2_ref[...]


def _tc_final(x_pad, z1, z2, psum3, inv, W3, l1w, lin1_b, ln_g, ln_b,
              lin2_w, lin2_b):
    grid = (N_PAD // BLK,)
    l0, l1, l2, l3 = (l1w[:, 0:D], l1w[:, D:2 * D],
                      l1w[:, 2 * D:3 * D], l1w[:, 3 * D:4 * D])
    row = pl.BlockSpec((BLK, D), lambda i: (i, 0))
    full = lambda a, b: pl.BlockSpec((a, b), lambda i: (0, 0))
    return pl.pallas_call(
        _tc_final_body,
        grid=grid,
        in_specs=[
            row, row, row,
            pl.BlockSpec((1, BLK, D), lambda i: (0, i, 0)),
            pl.BlockSpec((1, BLK, D), lambda i: (1, i, 0)),
            pl.BlockSpec((BLK, 1), lambda i: (i, 0)),
            full(D, D),
            full(HIDDEN, D), full(HIDDEN, D), full(HIDDEN, D), full(HIDDEN, D),
            full(1, HIDDEN), full(1, HIDDEN), full(1, HIDDEN),
            full(OUT_D, HIDDEN), full(1, OUT_D),
        ],
        out_specs=pl.BlockSpec((BLK, OUT_D), lambda i: (i, 0)),
        out_shape=jax.ShapeDtypeStruct((N_PAD, OUT_D), jnp.float32),
    )(x_pad, z1, z2, psum3, psum3, inv, W3, l0, l1, l2, l3,
      lin1_b.reshape(1, HIDDEN), ln_g.reshape(1, HIDDEN),
      ln_b.reshape(1, HIDDEN), lin2_w, lin2_b.reshape(1, OUT_D))


def kernel(x, edge_index, W1, W2, W3, lin1_w, lin1_b, ln_g, ln_b,
           lin2_w, lin2_b):
    ei = edge_index.astype(jnp.int32)
    src4 = ei[0].reshape(NW, NGRP, GRP, CHUNK)
    dst4 = ei[1].reshape(NW, NGRP, GRP, CHUNK)
    zrows = jnp.zeros((N_PAD, D), jnp.float32)
    ones_tab = jnp.ones((CHUNK, D), jnp.float32)

    x_pad = jnp.concatenate(
        [x, jnp.zeros((N_PAD - N_NODES, D), jnp.float32)], axis=0)

    psc = _sc_count(ones_tab, dst4, zrows)
    cnt = psc[0, :, 0:1] + psc[1, :, 0:1]
    inv = 1.0 / jnp.maximum(cnt, 1.0)

    psum1 = _sc_agg(x_pad, src4, dst4, zrows)
    z1 = _tc_layer(psum1, inv, W1)
    psum2 = _sc_agg(z1, src4, dst4, zrows)
    z2 = _tc_layer(psum2, inv, W2)
    psum3 = _sc_agg(z2, src4, dst4, zrows)
    out = _tc_final(x_pad, z1, z2, psum3, inv,
                    W3, lin1_w, lin1_b, ln_g, ln_b, lin2_w, lin2_b)
    return out[:N_NODES]


# speedup vs baseline: 8.6940x; 1.0115x over previous
import functools

import jax
import jax.numpy as jnp
from jax import lax
from jax.experimental import pallas as pl
from jax.experimental.pallas import tpu as pltpu
from jax.experimental.pallas import tpu_sc as plsc

N_NODES = 10000
N_EDGES = 320000
D = 128
HIDDEN = 256
OUT_D = 64
NC = 2
NS = 16
NW = NC * NS
E_PER_W = N_EDGES // NW
CHUNK = 100
GRP = 20
NGRP = 5
N_PAD = 10240
ROWS_PER_TILE = N_PAD // NS


def _sc_mesh():
    return plsc.VectorSubcoreMesh(
        core_axis_name="c", subcore_axis_name="s", num_cores=NC, num_subcores=NS
    )


def _make_sc_agg():
    scratch = [
        pltpu.VMEM_SHARED((N_PAD, D), jnp.float32),
        pltpu.VMEM((GRP, CHUNK), jnp.int32),
        pltpu.VMEM((GRP, CHUNK), jnp.int32),
        pltpu.VMEM((CHUNK, D), jnp.float32),
        pltpu.VMEM((CHUNK, D), jnp.float32),
        pltpu.SemaphoreType.DMA,
        pltpu.SemaphoreType.DMA,
        pltpu.SemaphoreType.DMA,
        pltpu.SemaphoreType.DMA,
    ]

    def body(table, src4, dst4, zrows, dummy, psum, acc_sh, src_v, dst_v,
             rows0, rows1, g0, g1, s0, s1):
        cid = lax.axis_index("c")
        sid = lax.axis_index("s")
        wid = cid * NS + sid
        r0 = sid * ROWS_PER_TILE
        rows = (rows0, rows1)
        gsem = (g0, g1)
        ssem = (s0, s1)

        pltpu.sync_copy(zrows.at[pl.ds(r0, ROWS_PER_TILE)],
                        acc_sh.at[pl.ds(r0, ROWS_PER_TILE)])
        plsc.subcore_barrier()

        tb = (GRP - 1) % 2

        def drain(sem, b):
            pltpu.make_async_copy(dummy, rows[b], sem).wait()

        def group_body(g, carry):
            pltpu.sync_copy(src4.at[wid, g], src_v)

            @pl.when(g > 0)
            def _():
                drain(s0, 0)
                drain(s1, 1)

            pltpu.sync_copy(dst4.at[wid, g], dst_v)

            for c in range(GRP):
                b = c % 2
                if c >= 2:
                    drain(ssem[b], b)
                pltpu.async_copy(table.at[src_v.at[c]], rows[b], gsem[b])
                if c >= 1:
                    drain(gsem[1 - b], 1 - b)
                    pltpu.async_copy(
                        rows[1 - b], acc_sh.at[dst_v.at[c - 1]], ssem[1 - b],
                        add=True)
            drain(gsem[tb], tb)
            pltpu.async_copy(rows[tb], acc_sh.at[dst_v.at[GRP - 1]],
                             ssem[tb], add=True)
            return carry

        lax.fori_loop(0, NGRP, group_body, 0)
        drain(s0, 0)
        drain(s1, 1)
        plsc.subcore_barrier()

        pltpu.sync_copy(acc_sh.at[pl.ds(r0, ROWS_PER_TILE)],
                        psum.at[cid, pl.ds(r0, ROWS_PER_TILE)])

    return pl.kernel(
        body, out_type=jax.ShapeDtypeStruct((NC, N_PAD, D), jnp.float32),
        mesh=_sc_mesh(), scratch_types=scratch,
    )


def _make_sc_count():
    scratch = [
        pltpu.VMEM_SHARED((N_PAD, D), jnp.float32),
        pltpu.VMEM((GRP, CHUNK), jnp.int32),
        pltpu.VMEM((CHUNK, D), jnp.float32),
        pltpu.SemaphoreType.DMA,
    ]

    def body(ones_tab, dst4, zrows, psum, acc_sh, dst_v, ones_v, sem):
        cid = lax.axis_index("c")
        sid = lax.axis_index("s")
        wid = cid * NS + sid
        r0 = sid * ROWS_PER_TILE

        pltpu.sync_copy(zrows.at[pl.ds(r0, ROWS_PER_TILE)],
                        acc_sh.at[pl.ds(r0, ROWS_PER_TILE)])
        pltpu.sync_copy(ones_tab.at[pl.ds(0, CHUNK)], ones_v)
        plsc.subcore_barrier()

        def group_body(g, carry):
            pltpu.sync_copy(dst4.at[wid, g], dst_v)
            descs = [
                pltpu.async_copy(ones_v, acc_sh.at[dst_v.at[c]], sem, add=True)
                for c in range(GRP)
            ]
            for d in descs:
                d.wait()
            return carry

        lax.fori_loop(0, NGRP, group_body, 0)
        plsc.subcore_barrier()
        pltpu.sync_copy(acc_sh.at[pl.ds(r0, ROWS_PER_TILE)],
                        psum.at[cid, pl.ds(r0, ROWS_PER_TILE)])

    return pl.kernel(
        body, out_type=jax.ShapeDtypeStruct((NC, N_PAD, D), jnp.float32),
        mesh=_sc_mesh(), scratch_types=scratch,
    )


_sc_agg = _make_sc_agg()
_sc_count = _make_sc_count()

BLK = 640


def _tc_layer_body(p0_ref, p1_ref, inv_ref, w_ref, o_ref):
    m = (p0_ref[0] + p1_ref[0]) * inv_ref[...]
    o_ref[...] = lax.dot_general(
        m, w_ref[...], (((1,), (1,)), ((), ())),
        preferred_element_type=jnp.float32)


def _tc_layer(psum, inv, W):
    grid = (N_PAD // BLK,)
    return pl.pallas_call(
        _tc_layer_body,
        grid=grid,
        in_specs=[
            pl.BlockSpec((1, BLK, D), lambda i: (0, i, 0)),
            pl.BlockSpec((1, BLK, D), lambda i: (1, i, 0)),
            pl.BlockSpec((BLK, 1), lambda i: (i, 0)),
            pl.BlockSpec((D, D), lambda i: (0, 0)),
        ],
        out_specs=pl.BlockSpec((BLK, D), lambda i: (i, 0)),
        out_shape=jax.ShapeDtypeStruct((N_PAD, D), jnp.float32),
    )(psum, psum, inv, W)


def _tc_final_body(x_ref, z1_ref, z2_ref, p30_ref, p31_ref, inv_ref, w3_ref,
                   l0_ref, l1_ref, l2_ref, l3_ref, b1_ref, g_ref, bb_ref,
                   l2w_ref, b2_ref, o_ref):
    dn = (((1,), (1,)), ((), ()))
    z3 = lax.dot_general(
        (p30_ref[0] + p31_ref[0]) * inv_ref[...], w3_ref[...], dn,
        preferred_element_type=jnp.float32)
    h = (lax.dot_general(x_ref[...], l0_ref[...], dn, preferred_element_type=jnp.float32)
         + lax.dot_general(z1_ref[...], l1_ref[...], dn, preferred_element_type=jnp.float32)
         + lax.dot_general(z2_ref[...], l2_ref[...], dn, preferred_element_type=jnp.float32)
         + lax.dot_general(z3, l3_ref[...], dn, preferred_element_type=jnp.float32)
         + b1_ref[...])
    mu = jnp.mean(h, axis=1, keepdims=True)
    var = jnp.mean((h - mu) ** 2, axis=1, keepdims=True)
    hn = (h - mu) * lax.rsqrt(var + 1e-5) * g_ref[...] + bb_ref[...]
    hr = jnp.maximum(hn, 0.0)
    o_ref[...] = lax.dot_general(
        hr, l2w_ref[...], dn, preferred_element_type=jnp.float32) + b2_ref[...]


def _tc_final(x_pad, z1, z2, psum3, inv, W3, l1w, lin1_b, ln_g, ln_b,
              lin2_w, lin2_b):
    grid = (N_PAD // BLK,)
    l0, l1, l2, l3 = (l1w[:, 0:D], l1w[:, D:2 * D],
                      l1w[:, 2 * D:3 * D], l1w[:, 3 * D:4 * D])
    row = pl.BlockSpec((BLK, D), lambda i: (i, 0))
    full = lambda a, b: pl.BlockSpec((a, b), lambda i: (0, 0))
    return pl.pallas_call(
        _tc_final_body,
        grid=grid,
        in_specs=[
            row, row, row,
            pl.BlockSpec((1, BLK, D), lambda i: (0, i, 0)),
            pl.BlockSpec((1, BLK, D), lambda i: (1, i, 0)),
            pl.BlockSpec((BLK, 1), lambda i: (i, 0)),
            full(D, D),
            full(HIDDEN, D), full(HIDDEN, D), full(HIDDEN, D), full(HIDDEN, D),
            full(1, HIDDEN), full(1, HIDDEN), full(1, HIDDEN),
            full(OUT_D, HIDDEN), full(1, OUT_D),
        ],
        out_specs=pl.BlockSpec((BLK, OUT_D), lambda i: (i, 0)),
        out_shape=jax.ShapeDtypeStruct((N_PAD, OUT_D), jnp.float32),
    )(x_pad, z1, z2, psum3, psum3, inv, W3, l0, l1, l2, l3,
      lin1_b.reshape(1, HIDDEN), ln_g.reshape(1, HIDDEN),
      ln_b.reshape(1, HIDDEN), lin2_w, lin2_b.reshape(1, OUT_D))


def kernel(x, edge_index, W1, W2, W3, lin1_w, lin1_b, ln_g, ln_b,
           lin2_w, lin2_b):
    ei = edge_index.astype(jnp.int32)
    src4 = ei[0].reshape(NW, NGRP, GRP, CHUNK)
    dst4 = ei[1].reshape(NW, NGRP, GRP, CHUNK)
    zrows = jnp.zeros((N_PAD, D), jnp.float32)
    ones_tab = jnp.ones((CHUNK, D), jnp.float32)

    x_pad = jnp.concatenate(
        [x, jnp.zeros((N_PAD - N_NODES, D), jnp.float32)], axis=0)

    psc = _sc_count(ones_tab, dst4, zrows)
    cnt = psc[0, :, 0:1] + psc[1, :, 0:1]
    inv = 1.0 / jnp.maximum(cnt, 1.0)

    psum1 = _sc_agg(x_pad, src4, dst4, zrows, ones_tab)
    z1 = _tc_layer(psum1, inv, W1)
    psum2 = _sc_agg(z1, src4, dst4, zrows, ones_tab)
    z2 = _tc_layer(psum2, inv, W2)
    psum3 = _sc_agg(z2, src4, dst4, zrows, ones_tab)
    out = _tc_final(x_pad, z1, z2, psum3, inv,
                    W3, lin1_w, lin1_b, ln_g, ln_b, lin2_w, lin2_b)
    return out[:N_NODES]
